# Initial kernel scaffold; baseline (speedup 1.0000x reference)
import jax
import jax.numpy as jnp
from jax.experimental import pallas as pl


def kernel(nodes, edges, conservation_scores, distances, Wq, bq, Wk, bk, Wv, bv, We, be, Wc1, bc1, Wc2, bc2, Wd1, bd1, Wd2, bd2, Wo, bo, gamma, beta, edge_index):
    raise NotImplementedError("write your pallas kernel here")


# speedup vs baseline: 849.6730x; 849.6730x over previous
import functools

import jax
import jax.numpy as jnp
from jax import lax
from jax.experimental import pallas as pl
from jax.experimental.pallas import tpu as pltpu
from jax.experimental.pallas import tpu_sc as plsc

_N = 10000
_E = 320000
_D = 128
_LANES = 16
_NC = 2
_NS = 16
_NW = _NC * _NS
_E_PER_W = _E // _NW
_N_PAD = 10240
_ROW_BLOCK = 1280


def _sc_mask_body(tgt_hbm, out_hbm, idx_v, mask_v):
    wid = lax.axis_index("s") * _NC + lax.axis_index("c")

    zeros = jnp.zeros((_LANES,), jnp.float32)
    ones = jnp.ones((_LANES,), jnp.float32)

    def zero_body(i, _):
        mask_v[pl.ds(i * _LANES, _LANES)] = zeros
        return 0

    lax.fori_loop(0, _N_PAD // _LANES, zero_body, 0)

    pltpu.sync_copy(tgt_hbm.at[pl.ds(wid * _E_PER_W, _E_PER_W)], idx_v)

    def scatter_body(i, _):
        idx16 = idx_v[pl.ds(i * _LANES, _LANES)]
        plsc.store_scatter(mask_v, [idx16], ones)
        return 0

    lax.fori_loop(0, _E_PER_W // _LANES, scatter_body, 0)

    pltpu.sync_copy(mask_v, out_hbm.at[wid])


@functools.cache
def _sc_mask():
    return pl.kernel(
        _sc_mask_body,
        out_type=jax.ShapeDtypeStruct((_NW, _N_PAD), jnp.float32),
        mesh=plsc.VectorSubcoreMesh(core_axis_name="c", subcore_axis_name="s"),
        scratch_types=[
            pltpu.VMEM((_E_PER_W,), jnp.int32),
            pltpu.VMEM((_N_PAD,), jnp.float32),
        ],
        compiler_params=pltpu.CompilerParams(
            use_tc_tiling_on_sc=False, needs_layout_passes=False),
    )


def _tc_fused_body(nodes_ref, part_ref, wv_ref, bv_ref, wo_ref, bo_ref,
                   g_ref, b_ref, out_ref):
    x = nodes_ref[...]
    cnt = jnp.sum(part_ref[...], axis=0)
    mask = (cnt > 0.0).astype(jnp.float32)[:, None]
    v = jnp.dot(x, wv_ref[...], preferred_element_type=jnp.float32) + bv_ref[...]
    out = jnp.dot(v * mask, wo_ref[...],
                  preferred_element_type=jnp.float32) + bo_ref[...]
    resid = x + out
    mean = jnp.mean(resid, axis=1, keepdims=True)
    cent = resid - mean
    var = jnp.mean(cent * cent, axis=1, keepdims=True)
    out_ref[...] = (cent / jnp.sqrt(var + 1e-5)) * g_ref[...] + b_ref[...]


def kernel(nodes, edges, conservation_scores, distances, Wq, bq, Wk, bk, Wv,
           bv, We, be, Wc1, bc1, Wc2, bc2, Wd1, bd1, Wd2, bd2, Wo, bo, gamma,
           beta, edge_index):
    tgt = edge_index[1]
    partials = _sc_mask()(tgt)

    grid = pl.cdiv(_N, _ROW_BLOCK)
    return pl.pallas_call(
        _tc_fused_body,
        grid=(grid,),
        in_specs=[
            pl.BlockSpec((_ROW_BLOCK, _D), lambda i: (i, 0)),
            pl.BlockSpec((_NW, _ROW_BLOCK), lambda i: (0, i)),
            pl.BlockSpec((_D, _D), lambda i: (0, 0)),
            pl.BlockSpec((1, _D), lambda i: (0, 0)),
            pl.BlockSpec((_D, _D), lambda i: (0, 0)),
            pl.BlockSpec((1, _D), lambda i: (0, 0)),
            pl.BlockSpec((1, _D), lambda i: (0, 0)),
            pl.BlockSpec((1, _D), lambda i: (0, 0)),
        ],
        out_specs=pl.BlockSpec((_ROW_BLOCK, _D), lambda i: (i, 0)),
        out_shape=jax.ShapeDtypeStruct((_N, _D), jnp.float32),
    )(nodes, partials, Wv, bv.reshape(1, _D), Wo, bo.reshape(1, _D),
      gamma.reshape(1, _D), beta.reshape(1, _D))


# speedup vs baseline: 1082.2902x; 1.2738x over previous
import functools

import jax
import jax.numpy as jnp
from jax import lax
from jax.experimental import pallas as pl
from jax.experimental.pallas import tpu as pltpu
from jax.experimental.pallas import tpu_sc as plsc

_N = 10000
_E = 320000
_D = 128
_LANES = 16
_NC = 2
_NS = 16
_NW = _NC * _NS
_E_PER_W = _E // _NW
_N_PAD = 10240
_ROW_BLOCK = 2560


_UNROLL = 5


def _sc_mask_body(edge_index_hbm, zeros_hbm, out_hbm, idx_v, mask_v):
    wid = lax.axis_index("s") * _NC + lax.axis_index("c")

    ones = jnp.ones((_LANES,), jnp.float32)

    pltpu.sync_copy(zeros_hbm, mask_v)
    pltpu.sync_copy(edge_index_hbm.at[1, pl.ds(wid * _E_PER_W, _E_PER_W)],
                    idx_v)

    def scatter_body(i, _):
        base = i * (_LANES * _UNROLL)
        for j in range(_UNROLL):
            idx16 = idx_v[pl.ds(base + j * _LANES, _LANES)]
            plsc.store_scatter(mask_v, [idx16], ones)
        return 0

    lax.fori_loop(0, _E_PER_W // (_LANES * _UNROLL), scatter_body, 0)

    pltpu.sync_copy(mask_v, out_hbm.at[wid])


@functools.cache
def _sc_mask():
    return pl.kernel(
        _sc_mask_body,
        out_type=jax.ShapeDtypeStruct((_NW, _N_PAD), jnp.float32),
        mesh=plsc.VectorSubcoreMesh(core_axis_name="c", subcore_axis_name="s"),
        scratch_types=[
            pltpu.VMEM((_E_PER_W,), jnp.int32),
            pltpu.VMEM((_N_PAD,), jnp.float32),
        ],
        compiler_params=pltpu.CompilerParams(
            use_tc_tiling_on_sc=False, needs_layout_passes=False),
    )


def _tc_fused_body(nodes_ref, part_ref, wv_ref, bv_ref, wo_ref, bo_ref,
                   g_ref, b_ref, out_ref):
    x = nodes_ref[...]
    cnt = jnp.sum(part_ref[...], axis=0)
    mask = (cnt > 0.0).astype(jnp.float32)[:, None]
    v = jnp.dot(x, wv_ref[...], preferred_element_type=jnp.float32) + bv_ref[...]
    out = jnp.dot(v * mask, wo_ref[...],
                  preferred_element_type=jnp.float32) + bo_ref[...]
    resid = x + out
    mean = jnp.mean(resid, axis=1, keepdims=True)
    cent = resid - mean
    var = jnp.mean(cent * cent, axis=1, keepdims=True)
    out_ref[...] = (cent / jnp.sqrt(var + 1e-5)) * g_ref[...] + b_ref[...]


def kernel(nodes, edges, conservation_scores, distances, Wq, bq, Wk, bk, Wv,
           bv, We, be, Wc1, bc1, Wc2, bc2, Wd1, bd1, Wd2, bd2, Wo, bo, gamma,
           beta, edge_index):
    partials = _sc_mask()(edge_index, jnp.zeros((_N_PAD,), jnp.float32))

    grid = pl.cdiv(_N, _ROW_BLOCK)
    return pl.pallas_call(
        _tc_fused_body,
        grid=(grid,),
        in_specs=[
            pl.BlockSpec((_ROW_BLOCK, _D), lambda i: (i, 0)),
            pl.BlockSpec((_NW, _ROW_BLOCK), lambda i: (0, i)),
            pl.BlockSpec((_D, _D), lambda i: (0, 0)),
            pl.BlockSpec((1, _D), lambda i: (0, 0)),
            pl.BlockSpec((_D, _D), lambda i: (0, 0)),
            pl.BlockSpec((1, _D), lambda i: (0, 0)),
            pl.BlockSpec((1, _D), lambda i: (0, 0)),
            pl.BlockSpec((1, _D), lambda i: (0, 0)),
        ],
        out_specs=pl.BlockSpec((_ROW_BLOCK, _D), lambda i: (i, 0)),
        out_shape=jax.ShapeDtypeStruct((_N, _D), jnp.float32),
    )(nodes, partials, Wv, bv.reshape(1, _D), Wo, bo.reshape(1, _D),
      gamma.reshape(1, _D), beta.reshape(1, _D))
